# baseline (speedup 1.0000x reference)
import jax
import jax.numpy as jnp
from jax import lax
from jax.experimental import pallas as pl
from jax.experimental.pallas import tpu as pltpu
from jax.experimental.pallas import tpu_sc as plsc

_D = 128
_N_I = 4096
_N_J = 50
_B_TOTAL = _N_I * _N_J
_NW = 32
_C = _N_I // _NW
_NCHUNK = _N_J
_NBUF = 12
_H = 64
_NHALF = _NCHUNK * 2


def _gather_body(idx_hbm, table_hbm, out_hbm, idx_v, bufs, gsem, ssem):
    cid = lax.axis_index("c")
    sid = lax.axis_index("s")
    wid = sid * 2 + cid
    c0 = wid * _C

    pltpu.sync_copy(idx_hbm.at[:, pl.ds(c0, _C)], idx_v)

    def gather(k, b):
        j, half = k // 2, k % 2
        src = table_hbm.at[idx_v.at[j, pl.ds(half * _H, _H)]]
        return pltpu.make_async_copy(src, bufs.at[b], gsem.at[b])

    def store(k, b):
        j, half = k // 2, k % 2
        dst = out_hbm.at[pl.ds(j * _N_I + c0 + half * _H, _H)]
        return pltpu.make_async_copy(bufs.at[b], dst, ssem.at[b])

    for b in range(_NBUF - 1):
        gather(b, b).start()

    def body(k, carry):
        b = lax.rem(k, _NBUF)
        kn = k + _NBUF - 1
        bn = lax.rem(kn, _NBUF)

        @pl.when(jnp.logical_and(kn < _NHALF, kn >= _NBUF))
        def _():
            store(kn - _NBUF, bn).wait()

        @pl.when(kn < _NHALF)
        def _():
            gather(kn, bn).start()

        gather(k, b).wait()
        store(k, b).start()
        return carry

    lax.fori_loop(0, _NHALF, body, None)

    for b in range(_NBUF):
        k_last = ((_NHALF - 1 - b) // _NBUF) * _NBUF + b
        store(k_last, b).wait()


_mesh = plsc.VectorSubcoreMesh(core_axis_name="c", subcore_axis_name="s")


@jax.jit
def _embed_lookup(idx_jmajor, table):
    return pl.kernel(
        _gather_body,
        out_type=jax.ShapeDtypeStruct((_B_TOTAL, _D), jnp.float32),
        mesh=_mesh,
        scratch_types=[
            pltpu.VMEM((_NCHUNK, _C), jnp.int32),
            pltpu.VMEM((_NBUF, _H, _D), jnp.float32),
            pltpu.SemaphoreType.DMA((_NBUF,)),
            pltpu.SemaphoreType.DMA((_NBUF,)),
        ],
        compiler_params=pltpu.CompilerParams(use_tc_tiling_on_sc=True),
    )(idx_jmajor, table)


def kernel(inputs, embedding):
    idx_jmajor = inputs.T.astype(jnp.int32)
    out = _embed_lookup(idx_jmajor, embedding)
    return out.reshape(_N_J, _N_I, _D).transpose(1, 0, 2)


# speedup vs baseline: 1.0087x; 1.0087x over previous
import jax
import jax.numpy as jnp
from jax import lax
from jax.experimental import pallas as pl
from jax.experimental.pallas import tpu as pltpu
from jax.experimental.pallas import tpu_sc as plsc

_D = 128
_N_I = 4096
_N_J = 50
_B_TOTAL = _N_I * _N_J
_NW = 32
_C = _N_I // _NW
_NCHUNK = _N_J
_NBUF = 7


def _gather_body(idx_hbm, table_hbm, out_hbm, idx_v, bufs, gsem, ssem):
    cid = lax.axis_index("c")
    sid = lax.axis_index("s")
    wid = sid * 2 + cid
    c0 = wid * _C

    pltpu.sync_copy(idx_hbm.at[:, pl.ds(c0, _C)], idx_v)

    def gather(j, b):
        src = table_hbm.at[idx_v.at[j]]
        return pltpu.make_async_copy(src, bufs.at[b], gsem.at[b])

    def store(j, b):
        dst = out_hbm.at[pl.ds(j * _N_I + c0, _C)]
        return pltpu.make_async_copy(bufs.at[b], dst, ssem.at[b])

    for b in range(_NBUF - 1):
        gather(b, b).start()

    def body(j, carry):
        b = lax.rem(j, _NBUF)
        jn = j + _NBUF - 1
        bn = lax.rem(jn, _NBUF)

        @pl.when(jnp.logical_and(jn < _NCHUNK, jn >= _NBUF))
        def _():
            store(jn - _NBUF, bn).wait()

        @pl.when(jn < _NCHUNK)
        def _():
            gather(jn, bn).start()

        gather(j, b).wait()
        store(j, b).start()
        return carry

    lax.fori_loop(0, _NCHUNK, body, None)

    for b in range(_NBUF):
        j_last = ((_NCHUNK - 1 - b) // _NBUF) * _NBUF + b
        store(j_last, b).wait()


_mesh = plsc.VectorSubcoreMesh(core_axis_name="c", subcore_axis_name="s")


@jax.jit
def _embed_lookup(idx_jmajor, table):
    return pl.kernel(
        _gather_body,
        out_type=jax.ShapeDtypeStruct((_B_TOTAL, _D), jnp.float32),
        mesh=_mesh,
        scratch_types=[
            pltpu.VMEM((_NCHUNK, _C), jnp.int32),
            pltpu.VMEM((_NBUF, _C, _D), jnp.float32),
            pltpu.SemaphoreType.DMA((_NBUF,)),
            pltpu.SemaphoreType.DMA((_NBUF,)),
        ],
        compiler_params=pltpu.CompilerParams(use_tc_tiling_on_sc=True),
    )(idx_jmajor, table)


def kernel(inputs, embedding):
    idx_jmajor = inputs.T.astype(jnp.int32)
    out = _embed_lookup(idx_jmajor, embedding)
    return out.reshape(_N_J, _N_I, _D).transpose(1, 0, 2)
